# Initial kernel scaffold; baseline (speedup 1.0000x reference)
import jax
import jax.numpy as jnp
from jax.experimental import pallas as pl


def kernel(x, router_w, router_b, imp_w, imp_b):
    raise NotImplementedError("write your pallas kernel here")


# speedup vs baseline: 2.2449x; 2.2449x over previous
import functools

import jax
import jax.numpy as jnp
from jax.experimental import pallas as pl
from jax.experimental.pallas import tpu as pltpu

_TOP_K = 8
_PAD_N = 128


def _router_kernel(x_ref, w_ref, b_ref,
                   probs_ref, idx_ref, wts_ref, loss_ref, imp_ref,
                   acc_ref, *, n_blocks, n_experts):
    i = pl.program_id(0)

    logits_full = jnp.dot(x_ref[...], w_ref[...],
                          preferred_element_type=jnp.float32) + b_ref[...]
    logits = logits_full[:, :n_experts]
    imp_ref[...] = jax.nn.sigmoid(logits_full[:, n_experts:n_experts + 1])

    m = jnp.max(logits, axis=-1, keepdims=True)
    ex = jnp.exp(logits - m)
    p = ex / jnp.sum(ex, axis=-1, keepdims=True)
    probs_ref[...] = p

    iota = jax.lax.broadcasted_iota(jnp.int32, p.shape, 1)
    work = p
    vals, idxs = [], []
    for _ in range(_TOP_K):
        mv = jnp.max(work, axis=-1, keepdims=True)
        ix = jnp.min(jnp.where(work == mv, iota, n_experts),
                     axis=-1, keepdims=True)
        vals.append(mv)
        idxs.append(ix)
        work = jnp.where(iota == ix, -jnp.inf, work)
    wts_ref[...] = jnp.concatenate(vals, axis=1)
    idx_ref[...] = jnp.concatenate(idxs, axis=1)

    masked = jnp.where(work == -jnp.inf, p, 0.0)
    colsum = jnp.sum(masked, axis=0, keepdims=True)

    @pl.when(i == 0)
    def _init():
        acc_ref[...] = jnp.zeros_like(acc_ref)

    acc_ref[...] += colsum

    @pl.when(i == n_blocks - 1)
    def _finalize():
        mask_sums = acc_ref[...]
        total = jnp.sum(mask_sums) + 1e-6
        em = mask_sums / total
        loss_ref[...] = jnp.sum(em * jnp.log(em + 1e-6),
                                keepdims=True).reshape(1, 1)


def kernel(x, router_w, router_b, imp_w, imp_b):
    n_tok, hidden = x.shape
    n_experts = router_w.shape[1]
    block = 512
    n_blocks = n_tok // block

    pad = _PAD_N - n_experts - 1
    w = jnp.concatenate(
        [router_w, imp_w, jnp.zeros((hidden, pad), x.dtype)], axis=1)
    b = jnp.concatenate(
        [router_b, imp_b, jnp.zeros((pad,), x.dtype)])[None, :]

    grid = (n_blocks,)
    probs, idx, wts, loss, imp = pl.pallas_call(
        functools.partial(_router_kernel, n_blocks=n_blocks,
                          n_experts=n_experts),
        grid=grid,
        in_specs=[
            pl.BlockSpec((block, hidden), lambda i: (i, 0)),
            pl.BlockSpec((hidden, _PAD_N), lambda i: (0, 0)),
            pl.BlockSpec((1, _PAD_N), lambda i: (0, 0)),
        ],
        out_specs=[
            pl.BlockSpec((block, n_experts), lambda i: (i, 0)),
            pl.BlockSpec((block, _TOP_K), lambda i: (i, 0)),
            pl.BlockSpec((block, _TOP_K), lambda i: (i, 0)),
            pl.BlockSpec((1, 1), lambda i: (0, 0)),
            pl.BlockSpec((block, 1), lambda i: (i, 0)),
        ],
        out_shape=[
            jax.ShapeDtypeStruct((n_tok, n_experts), jnp.float32),
            jax.ShapeDtypeStruct((n_tok, _TOP_K), jnp.int32),
            jax.ShapeDtypeStruct((n_tok, _TOP_K), jnp.float32),
            jax.ShapeDtypeStruct((1, 1), jnp.float32),
            jax.ShapeDtypeStruct((n_tok, 1), jnp.float32),
        ],
        scratch_shapes=[pltpu.VMEM((1, n_experts), jnp.float32)],
        compiler_params=pltpu.CompilerParams(
            dimension_semantics=("arbitrary",)),
    )(x, w, b)
    return probs, idx, wts, loss[0, 0], imp


# speedup vs baseline: 3.0240x; 1.3470x over previous
import functools

import jax
import jax.numpy as jnp
from jax.experimental import pallas as pl
from jax.experimental.pallas import tpu as pltpu

_TOP_K = 8
_PAD_N = 128


def _router_kernel(x_ref, wt_ref, bt_ref,
                   probs_ref, idx_ref, wts_ref, loss_ref, imp_ref,
                   acc_ref, *, n_blocks, n_experts, block):
    i = pl.program_id(0)

    lt_full = jax.lax.dot_general(
        wt_ref[...], x_ref[...], (((1,), (1,)), ((), ())),
        preferred_element_type=jnp.float32) + bt_ref[...]
    lt = lt_full[:n_experts, :]

    m = jnp.max(lt, axis=0, keepdims=True)
    ex = jnp.exp(lt - m)
    pt = ex * jax.lax.reciprocal(jnp.sum(ex, axis=0, keepdims=True))

    iota = jax.lax.broadcasted_iota(
        jnp.int32, pt.shape, 0).astype(jnp.float32)
    work = pt
    vals, idxs = [], []
    for _ in range(_TOP_K):
        mv = jnp.max(work, axis=0, keepdims=True)
        ixf = jnp.min(jnp.where(work == mv, iota, float(n_experts)),
                      axis=0, keepdims=True)
        vals.append(mv)
        idxs.append(ixf)
        work = jnp.where(iota == ixf, -jnp.inf, work)

    sig = jax.nn.sigmoid(lt_full[n_experts:n_experts + 1, :])
    pad_rows = _PAD_N - n_experts - 1
    out_t = jnp.concatenate(
        [pt, sig, jnp.zeros((pad_rows, block), jnp.float32)], axis=0)
    out = out_t.T
    probs_ref[...] = out[:, :n_experts]
    imp_ref[...] = out[:, n_experts:n_experts + 1]

    wts_ref[...] = jnp.concatenate(vals, axis=0).T
    idx_ref[...] = jnp.concatenate(idxs, axis=0).T.astype(jnp.int32)

    masked = jnp.where(work == -jnp.inf, pt, 0.0)

    @pl.when(i == 0)
    def _init():
        acc_ref[...] = jnp.zeros_like(acc_ref)

    acc_ref[...] += masked

    @pl.when(i == n_blocks - 1)
    def _finalize():
        mask_sums = jnp.sum(acc_ref[...], axis=1, keepdims=True)
        total = jnp.sum(mask_sums) + 1e-6
        em = mask_sums / total
        loss_ref[...] = jnp.sum(em * jnp.log(em + 1e-6),
                                keepdims=True).reshape(1, 1)


def kernel(x, router_w, router_b, imp_w, imp_b):
    n_tok, hidden = x.shape
    n_experts = router_w.shape[1]
    block = 512
    n_blocks = n_tok // block

    pad = _PAD_N - n_experts - 1
    wt = jnp.concatenate(
        [router_w, imp_w, jnp.zeros((hidden, pad), x.dtype)], axis=1).T
    bt = jnp.concatenate(
        [router_b, imp_b, jnp.zeros((pad,), x.dtype)])[:, None]

    grid = (n_blocks,)
    probs, idx, wts, loss, imp = pl.pallas_call(
        functools.partial(_router_kernel, n_blocks=n_blocks,
                          n_experts=n_experts, block=block),
        grid=grid,
        in_specs=[
            pl.BlockSpec((block, hidden), lambda i: (i, 0)),
            pl.BlockSpec((_PAD_N, hidden), lambda i: (0, 0)),
            pl.BlockSpec((_PAD_N, 1), lambda i: (0, 0)),
        ],
        out_specs=[
            pl.BlockSpec((block, n_experts), lambda i: (i, 0)),
            pl.BlockSpec((block, _TOP_K), lambda i: (i, 0)),
            pl.BlockSpec((block, _TOP_K), lambda i: (i, 0)),
            pl.BlockSpec((1, 1), lambda i: (0, 0)),
            pl.BlockSpec((block, 1), lambda i: (i, 0)),
        ],
        out_shape=[
            jax.ShapeDtypeStruct((n_tok, n_experts), jnp.float32),
            jax.ShapeDtypeStruct((n_tok, _TOP_K), jnp.int32),
            jax.ShapeDtypeStruct((n_tok, _TOP_K), jnp.float32),
            jax.ShapeDtypeStruct((1, 1), jnp.float32),
            jax.ShapeDtypeStruct((n_tok, 1), jnp.float32),
        ],
        scratch_shapes=[pltpu.VMEM((n_experts, block), jnp.float32)],
        compiler_params=pltpu.CompilerParams(
            dimension_semantics=("arbitrary",)),
    )(x, wt, bt)
    return probs, idx, wts, loss[0, 0], imp


# speedup vs baseline: 3.1672x; 1.0474x over previous
import functools

import jax
import jax.numpy as jnp
from jax.experimental import pallas as pl
from jax.experimental.pallas import tpu as pltpu

_TOP_K = 8
_PAD_N = 128


def _router_kernel(x_ref, wt_ref, bt_ref,
                   probs_ref, idx_ref, wts_ref, loss_ref, imp_ref,
                   acc_ref, *, n_blocks, n_experts, block):
    i = pl.program_id(0)

    lt_full = jax.lax.dot_general(
        wt_ref[...], x_ref[...], (((1,), (1,)), ((), ())),
        preferred_element_type=jnp.float32) + bt_ref[...]
    lt = lt_full[:n_experts, :]

    m = jnp.max(lt, axis=0, keepdims=True)
    ex = jnp.exp(lt - m)
    pt = ex * jax.lax.reciprocal(jnp.sum(ex, axis=0, keepdims=True))

    iota = jax.lax.broadcasted_iota(
        jnp.int32, pt.shape, 0).astype(jnp.float32)
    work = pt
    vals, idxs = [], []
    for _ in range(_TOP_K):
        mv = jnp.max(work, axis=0, keepdims=True)
        ixf = jnp.min(jnp.where(work == mv, iota, float(n_experts)),
                      axis=0, keepdims=True)
        vals.append(mv)
        idxs.append(ixf)
        work = jnp.where(iota == ixf, -jnp.inf, work)

    sig = jax.nn.sigmoid(lt_full[n_experts:n_experts + 1, :])
    pad_rows = _PAD_N - n_experts - 1
    out_t = jnp.concatenate(
        [pt, sig, jnp.zeros((pad_rows, block), jnp.float32)], axis=0)
    out = out_t.T
    probs_ref[...] = out[:, :n_experts]
    imp_ref[...] = out[:, n_experts:n_experts + 1]

    wts_ref[...] = jnp.concatenate(vals, axis=0).T
    idx_ref[...] = jnp.concatenate(idxs, axis=0).T.astype(jnp.int32)

    masked = jnp.where(work == -jnp.inf, pt, 0.0)

    @pl.when(i == 0)
    def _init():
        acc_ref[...] = jnp.zeros_like(acc_ref)

    acc_ref[...] += masked

    @pl.when(i == n_blocks - 1)
    def _finalize():
        mask_sums = jnp.sum(acc_ref[...], axis=1, keepdims=True)
        total = jnp.sum(mask_sums) + 1e-6
        em = mask_sums / total
        loss_ref[...] = jnp.sum(em * jnp.log(em + 1e-6),
                                keepdims=True).reshape(1, 1)


def kernel(x, router_w, router_b, imp_w, imp_b):
    n_tok, hidden = x.shape
    n_experts = router_w.shape[1]
    block = 1024
    n_blocks = n_tok // block

    pad = _PAD_N - n_experts - 1
    wt = jnp.concatenate(
        [router_w, imp_w, jnp.zeros((hidden, pad), x.dtype)], axis=1).T
    bt = jnp.concatenate(
        [router_b, imp_b, jnp.zeros((pad,), x.dtype)])[:, None]

    grid = (n_blocks,)
    probs, idx, wts, loss, imp = pl.pallas_call(
        functools.partial(_router_kernel, n_blocks=n_blocks,
                          n_experts=n_experts, block=block),
        grid=grid,
        in_specs=[
            pl.BlockSpec((block, hidden), lambda i: (i, 0)),
            pl.BlockSpec((_PAD_N, hidden), lambda i: (0, 0)),
            pl.BlockSpec((_PAD_N, 1), lambda i: (0, 0)),
        ],
        out_specs=[
            pl.BlockSpec((block, n_experts), lambda i: (i, 0)),
            pl.BlockSpec((block, _TOP_K), lambda i: (i, 0)),
            pl.BlockSpec((block, _TOP_K), lambda i: (i, 0)),
            pl.BlockSpec((1, 1), lambda i: (0, 0)),
            pl.BlockSpec((block, 1), lambda i: (i, 0)),
        ],
        out_shape=[
            jax.ShapeDtypeStruct((n_tok, n_experts), jnp.float32),
            jax.ShapeDtypeStruct((n_tok, _TOP_K), jnp.int32),
            jax.ShapeDtypeStruct((n_tok, _TOP_K), jnp.float32),
            jax.ShapeDtypeStruct((1, 1), jnp.float32),
            jax.ShapeDtypeStruct((n_tok, 1), jnp.float32),
        ],
        scratch_shapes=[pltpu.VMEM((n_experts, block), jnp.float32)],
        compiler_params=pltpu.CompilerParams(
            dimension_semantics=("arbitrary",)),
    )(x, wt, bt)
    return probs, idx, wts, loss[0, 0], imp
